# baseline (speedup 1.0000x reference)
import functools

import jax
import jax.numpy as jnp
from jax import lax
from jax.experimental import pallas as pl
from jax.experimental.pallas import tpu as pltpu
from jax.experimental.pallas import tpu_sc as plsc

_EMB = 128
_N = 100000
_T1_ROWS = 4 * 12 * 12
_T2_ROWS = 10 * 6 * 6 * 2 * 2

_NC, _NS = 2, 16
_NW = _NC * _NS
_B = 128
_NF = 9
_FULL_CHUNKS = _N // _B
_REM = _N - _FULL_CHUNKS * _B
_CHUNKS = _FULL_CHUNKS + 1
_NPAD = _CHUNKS * _B


def _build_tables_body(w1, w2, w3, w4, w5, w6, w7, w8, t1, t2):
    a = (w1[...][:, None, :] + w2[...][None, :, :]).reshape(48, _EMB)
    t1[...] = (a[:, None, :] + w3[...][None, :, :]).reshape(_T1_ROWS, _EMB)
    b = (w4[...][:, None, :] + w5[...][None, :, :]).reshape(60, _EMB)
    b = (b[:, None, :] + w6[...][None, :, :]).reshape(360, _EMB)
    b = (b[:, None, :] + w7[...][None, :, :]).reshape(720, _EMB)
    t2[...] = (b[:, None, :] + w8[...][None, :, :]).reshape(_T2_ROWS, _EMB)


def _build_tables(w1, w2, w3, w4, w5, w6, w7, w8):
    return pl.pallas_call(
        _build_tables_body,
        out_shape=[
            jax.ShapeDtypeStruct((_T1_ROWS, _EMB), jnp.float32),
            jax.ShapeDtypeStruct((_T2_ROWS, _EMB), jnp.float32),
        ],
    )(w1, w2, w3, w4, w5, w6, w7, w8)


def _fuse_idx_body(xt_ref, cc_ref):
    xb = xt_ref[...]
    c0 = xb[0:1, :]
    c1 = (xb[1:2, :] * 12 + xb[2:3, :]) * 12 + xb[3:4, :]
    c2 = (((xb[4:5, :] * 6 + xb[5:6, :]) * 6 + xb[6:7, :]) * 2
          + xb[7:8, :]) * 2 + xb[8:9, :]
    cc_ref[...] = jnp.concatenate([c0, c1, c2], axis=0).reshape(1, 3, _B)


def _fuse_idx(xt):
    return pl.pallas_call(
        _fuse_idx_body,
        grid=(_CHUNKS,),
        in_specs=[pl.BlockSpec((_NF, _B), lambda c: (0, c))],
        out_specs=pl.BlockSpec((1, 3, _B), lambda c: (c, 0, 0)),
        out_shape=jax.ShapeDtypeStruct((_CHUNKS, 3, _B), jnp.int32),
    )(xt)


def _stage_and_gather(c, ccp, w0, t1, t2, cv, r0, r1, r2, sem):
    off = pl.multiple_of(c * 3 * _B, 8)
    pltpu.sync_copy(ccp.at[pl.ds(off, 3 * _B)], cv)
    pltpu.async_copy(w0.at[cv.at[pl.ds(0, _B)]], r0, sem)
    pltpu.async_copy(t1.at[cv.at[pl.ds(_B, _B)]], r1, sem)
    pltpu.async_copy(t2.at[cv.at[pl.ds(2 * _B, _B)]], r2, sem)


def _wait_gathers(cv, w0, t1, t2, r0, r1, r2, sem):
    pltpu.make_async_copy(w0.at[cv.at[pl.ds(0, _B)]], r0, sem).wait()
    pltpu.make_async_copy(t1.at[cv.at[pl.ds(_B, _B)]], r1, sem).wait()
    pltpu.make_async_copy(t2.at[cv.at[pl.ds(2 * _B, _B)]], r2, sem).wait()


def _accumulate(r0, r1, r2, nrows):
    def acc_row(r, _):
        for l in range(_EMB // 16):
            sl = pl.ds(l * 16, 16)
            r0[r, sl] = r0[r, sl] + r1[r, sl] + r2[r, sl]
        return 0

    lax.fori_loop(0, nrows, acc_row, 0)


def _sc_lookup(ccp, w0, t1, t2):
    mesh = plsc.VectorSubcoreMesh(
        core_axis_name="c", subcore_axis_name="s",
        num_cores=_NC, num_subcores=_NS)

    @functools.partial(
        pl.kernel,
        out_type=jax.ShapeDtypeStruct((_N, _EMB), jnp.float32),
        mesh=mesh,
        scratch_types=dict(
            cv0=pltpu.VMEM((3 * _B,), jnp.int32),
            cv1=pltpu.VMEM((3 * _B,), jnp.int32),
            a0=pltpu.VMEM((_B, _EMB), jnp.float32),
            a1=pltpu.VMEM((_B, _EMB), jnp.float32),
            a2=pltpu.VMEM((_B, _EMB), jnp.float32),
            b0=pltpu.VMEM((_B, _EMB), jnp.float32),
            b1=pltpu.VMEM((_B, _EMB), jnp.float32),
            b2=pltpu.VMEM((_B, _EMB), jnp.float32),
            sem0=pltpu.SemaphoreType.DMA,
            sem1=pltpu.SemaphoreType.DMA,
        ),
    )
    def k(ccp_hbm, w0_hbm, t1_hbm, t2_hbm, out_hbm,
          cv0, cv1, a0, a1, a2, b0, b1, b2, sem0, sem1):
        wid = lax.axis_index("s") * _NC + lax.axis_index("c")
        nj = (_FULL_CHUNKS - wid + _NW - 1) // _NW

        def chunk_id(j):
            return wid + j * _NW

        def finish(c, cv, r0, r1, r2, sem):
            _wait_gathers(cv, w0_hbm, t1_hbm, t2_hbm, r0, r1, r2, sem)
            _accumulate(r0, r1, r2, _B)
            pltpu.sync_copy(r0, out_hbm.at[pl.ds(pl.multiple_of(c * _B, 8),
                                                 _B)])

        @pl.when(0 < nj)
        def _():
            _stage_and_gather(chunk_id(0), ccp_hbm, w0_hbm, t1_hbm, t2_hbm,
                              cv0, a0, a1, a2, sem0)

        def pair(jj, _):
            ja, jb, jc = 2 * jj, 2 * jj + 1, 2 * jj + 2

            @pl.when(jb < nj)
            def _():
                _stage_and_gather(chunk_id(jb), ccp_hbm, w0_hbm, t1_hbm,
                                  t2_hbm, cv1, b0, b1, b2, sem1)

            @pl.when(ja < nj)
            def _():
                finish(chunk_id(ja), cv0, a0, a1, a2, sem0)

            @pl.when(jc < nj)
            def _():
                _stage_and_gather(chunk_id(jc), ccp_hbm, w0_hbm, t1_hbm,
                                  t2_hbm, cv0, a0, a1, a2, sem0)

            @pl.when(jb < nj)
            def _():
                finish(chunk_id(jb), cv1, b0, b1, b2, sem1)

            return 0

        lax.fori_loop(0, (nj + 1) // 2, pair, 0)

        @pl.when(wid == _FULL_CHUNKS % _NW)
        def _():
            _stage_and_gather(_FULL_CHUNKS, ccp_hbm, w0_hbm, t1_hbm, t2_hbm,
                              cv0, a0, a1, a2, sem0)
            _wait_gathers(cv0, w0_hbm, t1_hbm, t2_hbm, a0, a1, a2, sem0)
            _accumulate(a0, a1, a2, _REM)
            pltpu.sync_copy(a0.at[pl.ds(0, _REM)],
                            out_hbm.at[pl.ds(_FULL_CHUNKS * _B, _REM)])

    return k(ccp, w0, t1, t2)


def kernel(x, W0, W1, W2, W3, W4, W5, W6, W7, W8):
    x32 = x.astype(jnp.int32)
    xt = jnp.pad(x32, ((0, _NPAD - _N), (0, 0))).T
    t1, t2 = _build_tables(W1, W2, W3, W4, W5, W6, W7, W8)
    ccp = _fuse_idx(xt).reshape(-1)
    return _sc_lookup(ccp, W0, t1, t2)


# speedup vs baseline: 5.3738x; 5.3738x over previous
import functools

import jax
import jax.numpy as jnp
from jax import lax
from jax.experimental import pallas as pl
from jax.experimental.pallas import tpu as pltpu
from jax.experimental.pallas import tpu_sc as plsc

_EMB = 128
_N = 100000
_T1_ROWS = 4 * 12 * 12
_T2_ROWS = 10 * 6 * 6 * 2 * 2

_NC, _NS = 2, 16
_NW = _NC * _NS
_B = 128
_NF = 9
_FULL_CHUNKS = _N // _B
_REM = _N - _FULL_CHUNKS * _B
_CHUNKS = _FULL_CHUNKS + 1
_NPAD = _CHUNKS * _B


def _build_tables_body(w1, w2, w3, w4, w5, w6, w7, w8, t1, t2):
    a = (w1[...][:, None, :] + w2[...][None, :, :]).reshape(48, _EMB)
    t1[...] = (a[:, None, :] + w3[...][None, :, :]).reshape(_T1_ROWS, _EMB)
    b = (w4[...][:, None, :] + w5[...][None, :, :]).reshape(60, _EMB)
    b = (b[:, None, :] + w6[...][None, :, :]).reshape(360, _EMB)
    b = (b[:, None, :] + w7[...][None, :, :]).reshape(720, _EMB)
    t2[...] = (b[:, None, :] + w8[...][None, :, :]).reshape(_T2_ROWS, _EMB)


def _build_tables(w1, w2, w3, w4, w5, w6, w7, w8):
    return pl.pallas_call(
        _build_tables_body,
        out_shape=[
            jax.ShapeDtypeStruct((_T1_ROWS, _EMB), jnp.float32),
            jax.ShapeDtypeStruct((_T2_ROWS, _EMB), jnp.float32),
        ],
    )(w1, w2, w3, w4, w5, w6, w7, w8)


def _fuse_idx_body(xt_ref, cc_ref):
    xb = xt_ref[...]
    c0 = xb[0:1, :]
    c1 = (xb[1:2, :] * 12 + xb[2:3, :]) * 12 + xb[3:4, :]
    c2 = (((xb[4:5, :] * 6 + xb[5:6, :]) * 6 + xb[6:7, :]) * 2
          + xb[7:8, :]) * 2 + xb[8:9, :]
    cc_ref[...] = jnp.concatenate([c0, c1, c2], axis=0).reshape(1, 3, _B)


def _fuse_idx(xt):
    return pl.pallas_call(
        _fuse_idx_body,
        grid=(_CHUNKS,),
        in_specs=[pl.BlockSpec((_NF, _B), lambda c: (0, c))],
        out_specs=pl.BlockSpec((1, 3, _B), lambda c: (c, 0, 0)),
        out_shape=jax.ShapeDtypeStruct((_CHUNKS, 3, _B), jnp.int32),
    )(xt)


def _stage_and_gather(c, ccp, w0, t1, t2, cv, r0, r1, r2, sem):
    off = pl.multiple_of(c * 3 * _B, 8)
    pltpu.sync_copy(ccp.at[pl.ds(off, 3 * _B)], cv)
    pltpu.async_copy(w0.at[cv.at[pl.ds(0, _B)]], r0, sem)
    pltpu.async_copy(t1.at[cv.at[pl.ds(_B, _B)]], r1, sem)
    pltpu.async_copy(t2.at[cv.at[pl.ds(2 * _B, _B)]], r2, sem)


def _wait_gathers(cv, w0, t1, t2, r0, r1, r2, sem):
    pltpu.make_async_copy(w0.at[cv.at[pl.ds(0, _B)]], r0, sem).wait()
    pltpu.make_async_copy(t1.at[cv.at[pl.ds(_B, _B)]], r1, sem).wait()
    pltpu.make_async_copy(t2.at[cv.at[pl.ds(2 * _B, _B)]], r2, sem).wait()


def _accumulate(r0, r1, r2, nrows):
    def acc_row(r, _):
        for l in range(_EMB // 16):
            sl = pl.ds(l * 16, 16)
            r0[r, sl] = r0[r, sl] + r1[r, sl] + r2[r, sl]
        return 0

    lax.fori_loop(0, nrows, acc_row, 0)


def _sc_lookup(ccp, w0, t1, t2):
    mesh = plsc.VectorSubcoreMesh(
        core_axis_name="c", subcore_axis_name="s",
        num_cores=_NC, num_subcores=_NS)

    @functools.partial(
        pl.kernel,
        out_type=jax.ShapeDtypeStruct((_N, _EMB), jnp.float32),
        mesh=mesh,
        scratch_types=dict(
            cv0=pltpu.VMEM((3 * _B,), jnp.int32),
            cv1=pltpu.VMEM((3 * _B,), jnp.int32),
            a0=pltpu.VMEM((_B, _EMB), jnp.float32),
            a1=pltpu.VMEM((_B, _EMB), jnp.float32),
            a2=pltpu.VMEM((_B, _EMB), jnp.float32),
            b0=pltpu.VMEM((_B, _EMB), jnp.float32),
            b1=pltpu.VMEM((_B, _EMB), jnp.float32),
            b2=pltpu.VMEM((_B, _EMB), jnp.float32),
            ts0=pltpu.VMEM_SHARED((119, _EMB), jnp.float32),
            ts1=pltpu.VMEM_SHARED((_T1_ROWS, _EMB), jnp.float32),
            ts2=pltpu.VMEM_SHARED((_T2_ROWS, _EMB), jnp.float32),
            sem0=pltpu.SemaphoreType.DMA,
            sem1=pltpu.SemaphoreType.DMA,
        ),
    )
    def k(ccp_hbm, w0_hbm, t1_hbm, t2_hbm, out_hbm,
          cv0, cv1, a0, a1, a2, b0, b1, b2, ts0, ts1, ts2, sem0, sem1):
        sid = lax.axis_index("s")
        wid = sid * _NC + lax.axis_index("c")

        @pl.when(sid == 0)
        def _():
            pltpu.sync_copy(w0_hbm, ts0)
            pltpu.sync_copy(t1_hbm, ts1)
            pltpu.sync_copy(t2_hbm, ts2)

        plsc.subcore_barrier()
        nj = (_FULL_CHUNKS - wid + _NW - 1) // _NW

        def chunk_id(j):
            return wid + j * _NW

        def finish(c, cv, r0, r1, r2, sem):
            _wait_gathers(cv, ts0, ts1, ts2, r0, r1, r2, sem)
            _accumulate(r0, r1, r2, _B)
            pltpu.sync_copy(r0, out_hbm.at[pl.ds(pl.multiple_of(c * _B, 8),
                                                 _B)])

        @pl.when(0 < nj)
        def _():
            _stage_and_gather(chunk_id(0), ccp_hbm, ts0, ts1, ts2,
                              cv0, a0, a1, a2, sem0)

        def pair(jj, _):
            ja, jb, jc = 2 * jj, 2 * jj + 1, 2 * jj + 2

            @pl.when(jb < nj)
            def _():
                _stage_and_gather(chunk_id(jb), ccp_hbm, ts0, ts1,
                                  ts2, cv1, b0, b1, b2, sem1)

            @pl.when(ja < nj)
            def _():
                finish(chunk_id(ja), cv0, a0, a1, a2, sem0)

            @pl.when(jc < nj)
            def _():
                _stage_and_gather(chunk_id(jc), ccp_hbm, ts0, ts1,
                                  ts2, cv0, a0, a1, a2, sem0)

            @pl.when(jb < nj)
            def _():
                finish(chunk_id(jb), cv1, b0, b1, b2, sem1)

            return 0

        lax.fori_loop(0, (nj + 1) // 2, pair, 0)

        @pl.when(wid == _FULL_CHUNKS % _NW)
        def _():
            _stage_and_gather(_FULL_CHUNKS, ccp_hbm, ts0, ts1, ts2,
                              cv0, a0, a1, a2, sem0)
            _wait_gathers(cv0, ts0, ts1, ts2, a0, a1, a2, sem0)
            _accumulate(a0, a1, a2, _REM)
            pltpu.sync_copy(a0.at[pl.ds(0, _REM)],
                            out_hbm.at[pl.ds(_FULL_CHUNKS * _B, _REM)])

    return k(ccp, w0, t1, t2)


def kernel(x, W0, W1, W2, W3, W4, W5, W6, W7, W8):
    x32 = x.astype(jnp.int32)
    xt = jnp.pad(x32, ((0, _NPAD - _N), (0, 0))).T
    t1, t2 = _build_tables(W1, W2, W3, W4, W5, W6, W7, W8)
    ccp = _fuse_idx(xt).reshape(-1)
    return _sc_lookup(ccp, W0, t1, t2)


# speedup vs baseline: 13.2267x; 2.4613x over previous
import functools

import jax
import jax.numpy as jnp
from jax import lax
from jax.experimental import pallas as pl
from jax.experimental.pallas import tpu as pltpu
from jax.experimental.pallas import tpu_sc as plsc

_EMB = 128
_N = 100000
_T1_ROWS = 4 * 12 * 12
_T2_ROWS = 10 * 6 * 6 * 2 * 2

_NC, _NS = 2, 16
_NW = _NC * _NS
_B = 128
_NF = 9
_FULL_CHUNKS = _N // _B
_REM = _N - _FULL_CHUNKS * _B
_CHUNKS = _FULL_CHUNKS + 1
_NPAD = _CHUNKS * _B


def _build_tables_body(w1, w2, w3, w4, w5, w6, w7, w8, t1, t2):
    a = (w1[...][:, None, :] + w2[...][None, :, :]).reshape(48, _EMB)
    t1[...] = (a[:, None, :] + w3[...][None, :, :]).reshape(_T1_ROWS, _EMB)
    b = (w4[...][:, None, :] + w5[...][None, :, :]).reshape(60, _EMB)
    b = (b[:, None, :] + w6[...][None, :, :]).reshape(360, _EMB)
    b = (b[:, None, :] + w7[...][None, :, :]).reshape(720, _EMB)
    t2[...] = (b[:, None, :] + w8[...][None, :, :]).reshape(_T2_ROWS, _EMB)


def _build_tables(w1, w2, w3, w4, w5, w6, w7, w8):
    return pl.pallas_call(
        _build_tables_body,
        out_shape=[
            jax.ShapeDtypeStruct((_T1_ROWS, _EMB), jnp.float32),
            jax.ShapeDtypeStruct((_T2_ROWS, _EMB), jnp.float32),
        ],
    )(w1, w2, w3, w4, w5, w6, w7, w8)


_FUSE_M = [
    [1, 0, 0, 0, 0, 0, 0, 0, 0],
    [0, 144, 12, 1, 0, 0, 0, 0, 0],
    [0, 0, 0, 0, 144, 24, 4, 2, 1],
]


def _prep_body(x_ref, m_ref, w1, w2, w3, w4, w5, w6, w7, w8, t1, t2, cc_ref):
    _build_tables_body(w1, w2, w3, w4, w5, w6, w7, w8, t1, t2)
    xf = x_ref[...].astype(jnp.float32)
    y = jax.lax.dot_general(m_ref[...], xf, (((1,), (1,)), ((), ())),
                            preferred_element_type=jnp.float32)
    yi = y.astype(jnp.int32).reshape(3, _CHUNKS, _B)
    cc_ref[...] = jnp.swapaxes(yi, 0, 1)


def _prep(xp, w1, w2, w3, w4, w5, w6, w7, w8):
    return pl.pallas_call(
        _prep_body,
        out_shape=[
            jax.ShapeDtypeStruct((_T1_ROWS, _EMB), jnp.float32),
            jax.ShapeDtypeStruct((_T2_ROWS, _EMB), jnp.float32),
            jax.ShapeDtypeStruct((_CHUNKS, 3, _B), jnp.int32),
        ],
    )(xp, jnp.asarray(_FUSE_M, dtype=jnp.float32),
      w1, w2, w3, w4, w5, w6, w7, w8)


def _stage_and_gather(c, ccp, w0, t1, t2, cv, r0, r1, r2, sem):
    off = pl.multiple_of(c * 3 * _B, 8)
    pltpu.sync_copy(ccp.at[pl.ds(off, 3 * _B)], cv)
    pltpu.async_copy(w0.at[cv.at[pl.ds(0, _B)]], r0, sem)
    pltpu.async_copy(t1.at[cv.at[pl.ds(_B, _B)]], r1, sem)
    pltpu.async_copy(t2.at[cv.at[pl.ds(2 * _B, _B)]], r2, sem)


def _wait_gathers(cv, w0, t1, t2, r0, r1, r2, sem):
    pltpu.make_async_copy(w0.at[cv.at[pl.ds(0, _B)]], r0, sem).wait()
    pltpu.make_async_copy(t1.at[cv.at[pl.ds(_B, _B)]], r1, sem).wait()
    pltpu.make_async_copy(t2.at[cv.at[pl.ds(2 * _B, _B)]], r2, sem).wait()


def _accumulate(r0, r1, r2, nrows):
    def acc_row(r, _):
        for l in range(_EMB // 16):
            sl = pl.ds(l * 16, 16)
            r0[r, sl] = r0[r, sl] + r1[r, sl] + r2[r, sl]
        return 0

    lax.fori_loop(0, nrows, acc_row, 0)


def _sc_lookup(ccp, w0, t1, t2):
    mesh = plsc.VectorSubcoreMesh(
        core_axis_name="c", subcore_axis_name="s",
        num_cores=_NC, num_subcores=_NS)

    @functools.partial(
        pl.kernel,
        out_type=jax.ShapeDtypeStruct((_N, _EMB), jnp.float32),
        mesh=mesh,
        scratch_types=dict(
            cv0=pltpu.VMEM((3 * _B,), jnp.int32),
            cv1=pltpu.VMEM((3 * _B,), jnp.int32),
            a0=pltpu.VMEM((_B, _EMB), jnp.float32),
            a1=pltpu.VMEM((_B, _EMB), jnp.float32),
            a2=pltpu.VMEM((_B, _EMB), jnp.float32),
            b0=pltpu.VMEM((_B, _EMB), jnp.float32),
            b1=pltpu.VMEM((_B, _EMB), jnp.float32),
            b2=pltpu.VMEM((_B, _EMB), jnp.float32),
            ts0=pltpu.VMEM_SHARED((119, _EMB), jnp.float32),
            ts1=pltpu.VMEM_SHARED((_T1_ROWS, _EMB), jnp.float32),
            ts2=pltpu.VMEM_SHARED((_T2_ROWS, _EMB), jnp.float32),
            sem0=pltpu.SemaphoreType.DMA,
            sem1=pltpu.SemaphoreType.DMA,
        ),
    )
    def k(ccp_hbm, w0_hbm, t1_hbm, t2_hbm, out_hbm,
          cv0, cv1, a0, a1, a2, b0, b1, b2, ts0, ts1, ts2, sem0, sem1):
        sid = lax.axis_index("s")
        wid = sid * _NC + lax.axis_index("c")

        @pl.when(sid == 0)
        def _():
            pltpu.sync_copy(w0_hbm, ts0)
            pltpu.sync_copy(t1_hbm, ts1)
            pltpu.sync_copy(t2_hbm, ts2)

        plsc.subcore_barrier()
        nj = (_FULL_CHUNKS - wid + _NW - 1) // _NW

        def chunk_id(j):
            return wid + j * _NW

        def finish(c, cv, r0, r1, r2, sem):
            _wait_gathers(cv, ts0, ts1, ts2, r0, r1, r2, sem)
            _accumulate(r0, r1, r2, _B)
            pltpu.sync_copy(r0, out_hbm.at[pl.ds(pl.multiple_of(c * _B, 8),
                                                 _B)])

        @pl.when(0 < nj)
        def _():
            _stage_and_gather(chunk_id(0), ccp_hbm, ts0, ts1, ts2,
                              cv0, a0, a1, a2, sem0)

        def pair(jj, _):
            ja, jb, jc = 2 * jj, 2 * jj + 1, 2 * jj + 2

            @pl.when(jb < nj)
            def _():
                _stage_and_gather(chunk_id(jb), ccp_hbm, ts0, ts1,
                                  ts2, cv1, b0, b1, b2, sem1)

            @pl.when(ja < nj)
            def _():
                finish(chunk_id(ja), cv0, a0, a1, a2, sem0)

            @pl.when(jc < nj)
            def _():
                _stage_and_gather(chunk_id(jc), ccp_hbm, ts0, ts1,
                                  ts2, cv0, a0, a1, a2, sem0)

            @pl.when(jb < nj)
            def _():
                finish(chunk_id(jb), cv1, b0, b1, b2, sem1)

            return 0

        lax.fori_loop(0, (nj + 1) // 2, pair, 0)

        @pl.when(wid == _FULL_CHUNKS % _NW)
        def _():
            _stage_and_gather(_FULL_CHUNKS, ccp_hbm, ts0, ts1, ts2,
                              cv0, a0, a1, a2, sem0)
            _wait_gathers(cv0, ts0, ts1, ts2, a0, a1, a2, sem0)
            _accumulate(a0, a1, a2, _REM)
            pltpu.sync_copy(a0.at[pl.ds(0, _REM)],
                            out_hbm.at[pl.ds(_FULL_CHUNKS * _B, _REM)])

    return k(ccp, w0, t1, t2)


def kernel(x, W0, W1, W2, W3, W4, W5, W6, W7, W8):
    xp = jnp.pad(x.astype(jnp.int32), ((0, _NPAD - _N), (0, 0)))
    t1, t2, ccp3 = _prep(xp, W1, W2, W3, W4, W5, W6, W7, W8)
    return _sc_lookup(ccp3.reshape(-1), W0, t1, t2)


# speedup vs baseline: 15.9050x; 1.2025x over previous
import functools

import jax
import jax.numpy as jnp
from jax import lax
from jax.experimental import pallas as pl
from jax.experimental.pallas import tpu as pltpu
from jax.experimental.pallas import tpu_sc as plsc

_EMB = 128
_N = 100000
_T1_ROWS = 4 * 12 * 12
_T2_ROWS = 10 * 6 * 6 * 2 * 2

_NC, _NS = 2, 16
_NW = _NC * _NS
_B = 128
_NF = 9
_FULL_CHUNKS = _N // _B
_REM = _N - _FULL_CHUNKS * _B
_CHUNKS = _FULL_CHUNKS + 1
_NPAD = _CHUNKS * _B


def _build_tables_body(w1, w2, w3, w4, w5, w6, w7, w8, t1, t2):
    a = (w1[...][:, None, :] + w2[...][None, :, :]).reshape(48, _EMB)
    t1[...] = (a[:, None, :] + w3[...][None, :, :]).reshape(_T1_ROWS, _EMB)
    b = (w4[...][:, None, :] + w5[...][None, :, :]).reshape(60, _EMB)
    b = (b[:, None, :] + w6[...][None, :, :]).reshape(360, _EMB)
    b = (b[:, None, :] + w7[...][None, :, :]).reshape(720, _EMB)
    t2[...] = (b[:, None, :] + w8[...][None, :, :]).reshape(_T2_ROWS, _EMB)


def _build_tables(w1, w2, w3, w4, w5, w6, w7, w8):
    return pl.pallas_call(
        _build_tables_body,
        out_shape=[
            jax.ShapeDtypeStruct((_T1_ROWS, _EMB), jnp.float32),
            jax.ShapeDtypeStruct((_T2_ROWS, _EMB), jnp.float32),
        ],
    )(w1, w2, w3, w4, w5, w6, w7, w8)


_FUSE_M = [
    [1, 0, 0, 0, 0, 0, 0, 0, 0],
    [0, 144, 12, 1, 0, 0, 0, 0, 0],
    [0, 0, 0, 0, 144, 24, 4, 2, 1],
]


def _prep_body(x_ref, m_ref, w1, w2, w3, w4, w5, w6, w7, w8, t1, t2, cc_ref):
    _build_tables_body(w1, w2, w3, w4, w5, w6, w7, w8, t1, t2)
    xf = x_ref[...].astype(jnp.float32)
    xf = jnp.concatenate(
        [xf, jnp.zeros((_NPAD - _N, _NF), jnp.float32)], axis=0)
    y = jax.lax.dot_general(m_ref[...], xf, (((1,), (1,)), ((), ())),
                            preferred_element_type=jnp.float32)
    yi = y.astype(jnp.int32).reshape(3, _CHUNKS, _B)
    cc_ref[...] = jnp.swapaxes(yi, 0, 1)


def _prep(xp, w1, w2, w3, w4, w5, w6, w7, w8):
    return pl.pallas_call(
        _prep_body,
        out_shape=[
            jax.ShapeDtypeStruct((_T1_ROWS, _EMB), jnp.float32),
            jax.ShapeDtypeStruct((_T2_ROWS, _EMB), jnp.float32),
            jax.ShapeDtypeStruct((_CHUNKS, 3, _B), jnp.int32),
        ],
    )(xp, jnp.asarray(_FUSE_M, dtype=jnp.float32),
      w1, w2, w3, w4, w5, w6, w7, w8)


def _stage_and_gather(c, ccp, w0, t1, t2, cv, r0, r1, r2, sem):
    off = pl.multiple_of(c * 3 * _B, 8)
    pltpu.sync_copy(ccp.at[pl.ds(off, 3 * _B)], cv)
    pltpu.async_copy(w0.at[cv.at[pl.ds(0, _B)]], r0, sem)
    pltpu.async_copy(t1.at[cv.at[pl.ds(_B, _B)]], r1, sem)
    pltpu.async_copy(t2.at[cv.at[pl.ds(2 * _B, _B)]], r2, sem)


def _wait_gathers(cv, w0, t1, t2, r0, r1, r2, sem):
    pltpu.make_async_copy(w0.at[cv.at[pl.ds(0, _B)]], r0, sem).wait()
    pltpu.make_async_copy(t1.at[cv.at[pl.ds(_B, _B)]], r1, sem).wait()
    pltpu.make_async_copy(t2.at[cv.at[pl.ds(2 * _B, _B)]], r2, sem).wait()


def _accumulate(r0, r1, r2, nrows):
    def acc_row(r):
        for l in range(_EMB // 16):
            sl = pl.ds(l * 16, 16)
            r0[r, sl] = r0[r, sl] + r1[r, sl] + r2[r, sl]

    plsc.parallel_loop(0, nrows, 1, unroll=4)(acc_row)


def _sc_lookup(ccp, w0, t1, t2):
    mesh = plsc.VectorSubcoreMesh(
        core_axis_name="c", subcore_axis_name="s",
        num_cores=_NC, num_subcores=_NS)

    @functools.partial(
        pl.kernel,
        out_type=jax.ShapeDtypeStruct((_N, _EMB), jnp.float32),
        mesh=mesh,
        scratch_types=dict(
            cv0=pltpu.VMEM((3 * _B,), jnp.int32),
            cv1=pltpu.VMEM((3 * _B,), jnp.int32),
            a0=pltpu.VMEM((_B, _EMB), jnp.float32),
            a1=pltpu.VMEM((_B, _EMB), jnp.float32),
            a2=pltpu.VMEM((_B, _EMB), jnp.float32),
            b0=pltpu.VMEM((_B, _EMB), jnp.float32),
            b1=pltpu.VMEM((_B, _EMB), jnp.float32),
            b2=pltpu.VMEM((_B, _EMB), jnp.float32),
            ts0=pltpu.VMEM_SHARED((119, _EMB), jnp.float32),
            ts1=pltpu.VMEM_SHARED((_T1_ROWS, _EMB), jnp.float32),
            ts2=pltpu.VMEM_SHARED((_T2_ROWS, _EMB), jnp.float32),
            sem0=pltpu.SemaphoreType.DMA,
            sem1=pltpu.SemaphoreType.DMA,
        ),
    )
    def k(ccp_hbm, w0_hbm, t1_hbm, t2_hbm, out_hbm,
          cv0, cv1, a0, a1, a2, b0, b1, b2, ts0, ts1, ts2, sem0, sem1):
        sid = lax.axis_index("s")
        wid = sid * _NC + lax.axis_index("c")

        @pl.when(sid == 0)
        def _():
            pltpu.sync_copy(w0_hbm, ts0)
            pltpu.sync_copy(t1_hbm, ts1)
            pltpu.sync_copy(t2_hbm, ts2)

        plsc.subcore_barrier()
        nj = (_FULL_CHUNKS - wid + _NW - 1) // _NW

        def chunk_id(j):
            return wid + j * _NW

        def finish(c, cv, r0, r1, r2, sem):
            _wait_gathers(cv, ts0, ts1, ts2, r0, r1, r2, sem)
            _accumulate(r0, r1, r2, _B)
            pltpu.sync_copy(r0, out_hbm.at[pl.ds(pl.multiple_of(c * _B, 8),
                                                 _B)])

        @pl.when(0 < nj)
        def _():
            _stage_and_gather(chunk_id(0), ccp_hbm, ts0, ts1, ts2,
                              cv0, a0, a1, a2, sem0)

        def pair(jj, _):
            ja, jb, jc = 2 * jj, 2 * jj + 1, 2 * jj + 2

            @pl.when(jb < nj)
            def _():
                _stage_and_gather(chunk_id(jb), ccp_hbm, ts0, ts1,
                                  ts2, cv1, b0, b1, b2, sem1)

            @pl.when(ja < nj)
            def _():
                finish(chunk_id(ja), cv0, a0, a1, a2, sem0)

            @pl.when(jc < nj)
            def _():
                _stage_and_gather(chunk_id(jc), ccp_hbm, ts0, ts1,
                                  ts2, cv0, a0, a1, a2, sem0)

            @pl.when(jb < nj)
            def _():
                finish(chunk_id(jb), cv1, b0, b1, b2, sem1)

            return 0

        lax.fori_loop(0, (nj + 1) // 2, pair, 0)

        @pl.when(wid == _FULL_CHUNKS % _NW)
        def _():
            _stage_and_gather(_FULL_CHUNKS, ccp_hbm, ts0, ts1, ts2,
                              cv0, a0, a1, a2, sem0)
            _wait_gathers(cv0, ts0, ts1, ts2, a0, a1, a2, sem0)
            _accumulate(a0, a1, a2, _REM)
            pltpu.sync_copy(a0.at[pl.ds(0, _REM)],
                            out_hbm.at[pl.ds(_FULL_CHUNKS * _B, _REM)])

    return k(ccp, w0, t1, t2)


def kernel(x, W0, W1, W2, W3, W4, W5, W6, W7, W8):
    t1, t2, ccp3 = _prep(x.astype(jnp.int32), W1, W2, W3, W4, W5, W6, W7, W8)
    return _sc_lookup(ccp3.reshape(-1), W0, t1, t2)


# speedup vs baseline: 16.2785x; 1.0235x over previous
import functools

import jax
import jax.numpy as jnp
from jax import lax
from jax.experimental import pallas as pl
from jax.experimental.pallas import tpu as pltpu
from jax.experimental.pallas import tpu_sc as plsc

_EMB = 128
_N = 100000
_T1_ROWS = 4 * 12 * 12
_T2_ROWS = 10 * 6 * 6 * 2 * 2

_NC, _NS = 2, 16
_NW = _NC * _NS
_B = 128
_NF = 9
_FULL_CHUNKS = _N // _B
_REM = _N - _FULL_CHUNKS * _B
_CHUNKS = _FULL_CHUNKS + 1
_NPAD = _CHUNKS * _B


def _build_tables_body(w1, w2, w3, w4, w5, w6, w7, w8, t1, t2):
    a = (w1[...][:, None, :] + w2[...][None, :, :]).reshape(48, _EMB)
    t1[...] = (a[:, None, :] + w3[...][None, :, :]).reshape(_T1_ROWS, _EMB)
    b = (w4[...][:, None, :] + w5[...][None, :, :]).reshape(60, _EMB)
    b = (b[:, None, :] + w6[...][None, :, :]).reshape(360, _EMB)
    b = (b[:, None, :] + w7[...][None, :, :]).reshape(720, _EMB)
    t2[...] = (b[:, None, :] + w8[...][None, :, :]).reshape(_T2_ROWS, _EMB)


def _build_tables(w1, w2, w3, w4, w5, w6, w7, w8):
    return pl.pallas_call(
        _build_tables_body,
        out_shape=[
            jax.ShapeDtypeStruct((_T1_ROWS, _EMB), jnp.float32),
            jax.ShapeDtypeStruct((_T2_ROWS, _EMB), jnp.float32),
        ],
    )(w1, w2, w3, w4, w5, w6, w7, w8)


_FUSE_M = [
    [1, 0, 0, 0, 0, 0, 0, 0, 0],
    [0, 144, 12, 1, 0, 0, 0, 0, 0],
    [0, 0, 0, 0, 144, 24, 4, 2, 1],
]


def _prep_body(x_ref, m_ref, w1, w2, w3, w4, w5, w6, w7, w8, t1, t2, cc_ref):
    _build_tables_body(w1, w2, w3, w4, w5, w6, w7, w8, t1, t2)
    xf = x_ref[...].astype(jnp.float32)
    xf = jnp.concatenate(
        [xf, jnp.zeros((_NPAD - _N, _NF), jnp.float32)], axis=0)
    y = jax.lax.dot_general(m_ref[...], xf, (((1,), (1,)), ((), ())),
                            preferred_element_type=jnp.float32)
    yi = y.astype(jnp.int32).reshape(3, _CHUNKS, _B)
    cc_ref[...] = jnp.swapaxes(yi, 0, 1)


def _prep(xp, w1, w2, w3, w4, w5, w6, w7, w8):
    return pl.pallas_call(
        _prep_body,
        out_shape=[
            jax.ShapeDtypeStruct((_T1_ROWS, _EMB), jnp.float32),
            jax.ShapeDtypeStruct((_T2_ROWS, _EMB), jnp.float32),
            jax.ShapeDtypeStruct((_CHUNKS, 3, _B), jnp.int32),
        ],
    )(xp, jnp.asarray(_FUSE_M, dtype=jnp.float32),
      w1, w2, w3, w4, w5, w6, w7, w8)


def _start_gathers(j, w0, t1, t2, cva, r0, r1, r2, sem):
    base = j * 3 * _B
    pltpu.async_copy(w0.at[cva.at[pl.ds(base, _B)]], r0, sem)
    pltpu.async_copy(t1.at[cva.at[pl.ds(base + _B, _B)]], r1, sem)
    pltpu.async_copy(t2.at[cva.at[pl.ds(base + 2 * _B, _B)]], r2, sem)


def _wait_gathers(j, w0, t1, t2, cva, r0, r1, r2, sem):
    base = j * 3 * _B
    pltpu.make_async_copy(w0.at[cva.at[pl.ds(base, _B)]], r0, sem).wait()
    pltpu.make_async_copy(t1.at[cva.at[pl.ds(base + _B, _B)]], r1,
                          sem).wait()
    pltpu.make_async_copy(t2.at[cva.at[pl.ds(base + 2 * _B, _B)]], r2,
                          sem).wait()


def _accumulate(r0, r1, r2, nrows):
    def acc_row(r):
        for l in range(_EMB // 16):
            sl = pl.ds(l * 16, 16)
            r0[r, sl] = r0[r, sl] + r1[r, sl] + r2[r, sl]

    plsc.parallel_loop(0, nrows, 1, unroll=4)(acc_row)


def _sc_lookup(ccp, w0, t1, t2):
    mesh = plsc.VectorSubcoreMesh(
        core_axis_name="c", subcore_axis_name="s",
        num_cores=_NC, num_subcores=_NS)

    @functools.partial(
        pl.kernel,
        out_type=jax.ShapeDtypeStruct((_N, _EMB), jnp.float32),
        mesh=mesh,
        scratch_types=dict(
            cva=pltpu.VMEM((25 * 3 * _B,), jnp.int32),
            a0=pltpu.VMEM((_B, _EMB), jnp.float32),
            a1=pltpu.VMEM((_B, _EMB), jnp.float32),
            a2=pltpu.VMEM((_B, _EMB), jnp.float32),
            b0=pltpu.VMEM((_B, _EMB), jnp.float32),
            b1=pltpu.VMEM((_B, _EMB), jnp.float32),
            b2=pltpu.VMEM((_B, _EMB), jnp.float32),
            ts0=pltpu.VMEM_SHARED((119, _EMB), jnp.float32),
            ts1=pltpu.VMEM_SHARED((_T1_ROWS, _EMB), jnp.float32),
            ts2=pltpu.VMEM_SHARED((_T2_ROWS, _EMB), jnp.float32),
            sem0=pltpu.SemaphoreType.DMA,
            sem1=pltpu.SemaphoreType.DMA,
            isem=pltpu.SemaphoreType.DMA,
            wsem0=pltpu.SemaphoreType.DMA,
            wsem1=pltpu.SemaphoreType.DMA,
        ),
    )
    def k(ccp_hbm, w0_hbm, t1_hbm, t2_hbm, out_hbm,
          cva, a0, a1, a2, b0, b1, b2, ts0, ts1, ts2,
          sem0, sem1, isem, wsem0, wsem1):
        sid = lax.axis_index("s")
        wid = sid * _NC + lax.axis_index("c")

        @pl.when(sid == 0)
        def _():
            pltpu.sync_copy(w0_hbm, ts0)
            pltpu.sync_copy(t1_hbm, ts1)
            pltpu.sync_copy(t2_hbm, ts2)

        plsc.subcore_barrier()
        nj = (_FULL_CHUNKS - wid + _NW - 1) // _NW

        def chunk_id(j):
            return wid + j * _NW

        def stage_idx(j, _):
            off = pl.multiple_of(chunk_id(j) * 3 * _B, 8)
            pltpu.async_copy(ccp_hbm.at[pl.ds(off, 3 * _B)],
                             cva.at[pl.ds(j * 3 * _B, 3 * _B)], isem)
            return 0

        def wait_idx(j, _):
            pltpu.make_async_copy(ccp_hbm.at[pl.ds(0, 3 * _B)],
                                  cva.at[pl.ds(0, 3 * _B)], isem).wait()
            return 0

        lax.fori_loop(0, nj, stage_idx, 0)
        lax.fori_loop(0, nj, wait_idx, 0)

        def wb(c, r0, wsem):
            pltpu.async_copy(
                r0, out_hbm.at[pl.ds(pl.multiple_of(c * _B, 8), _B)], wsem)

        def drain_wb(r0, wsem):
            pltpu.make_async_copy(r0, out_hbm.at[pl.ds(0, _B)], wsem).wait()

        def finish(j, r0, r1, r2, sem, wsem):
            _wait_gathers(j, ts0, ts1, ts2, cva, r0, r1, r2, sem)
            _accumulate(r0, r1, r2, _B)
            wb(chunk_id(j), r0, wsem)

        _start_gathers(0, ts0, ts1, ts2, cva, a0, a1, a2, sem0)

        def pair(jj, _):
            ja, jb, jc = 2 * jj, 2 * jj + 1, 2 * jj + 2

            @pl.when(jb < nj)
            def _():
                @pl.when(jj > 0)
                def _():
                    drain_wb(b0, wsem1)

                _start_gathers(jb, ts0, ts1, ts2, cva, b0, b1, b2, sem1)

            @pl.when(ja < nj)
            def _():
                finish(ja, a0, a1, a2, sem0, wsem0)

            @pl.when(jc < nj)
            def _():
                drain_wb(a0, wsem0)
                _start_gathers(jc, ts0, ts1, ts2, cva, a0, a1, a2, sem0)

            @pl.when(jb < nj)
            def _():
                finish(jb, b0, b1, b2, sem1, wsem1)

            return 0

        lax.fori_loop(0, (nj + 1) // 2, pair, 0)
        drain_wb(a0, wsem0)
        drain_wb(b0, wsem1)

        @pl.when(wid == _FULL_CHUNKS % _NW)
        def _():
            pltpu.sync_copy(ccp_hbm.at[pl.ds(_FULL_CHUNKS * 3 * _B, 3 * _B)],
                            cva.at[pl.ds(24 * 3 * _B, 3 * _B)])
            _start_gathers(24, ts0, ts1, ts2, cva, a0, a1, a2, sem0)
            _wait_gathers(24, ts0, ts1, ts2, cva, a0, a1, a2, sem0)
            _accumulate(a0, a1, a2, _REM)
            pltpu.sync_copy(a0.at[pl.ds(0, _REM)],
                            out_hbm.at[pl.ds(_FULL_CHUNKS * _B, _REM)])

    return k(ccp, w0, t1, t2)


def kernel(x, W0, W1, W2, W3, W4, W5, W6, W7, W8):
    t1, t2, ccp3 = _prep(x.astype(jnp.int32), W1, W2, W3, W4, W5, W6, W7, W8)
    return _sc_lookup(ccp3.reshape(-1), W0, t1, t2)
